# baseline (speedup 1.0000x reference)
import math

import jax
import jax.numpy as jnp
from jax.experimental import pallas as pl
from jax.experimental.pallas import tpu as pltpu

B = 1
S = 2048
D_MODEL = 768
H = 12
DH = D_MODEL // H
BS = 256
NB = S // BS
NPRI = NB - 1
PRI = NPRI * BS
TOPK = 3
SCALE = 1.0 / math.sqrt(DH)
INV_GATE = 1.0 / (BS * SCALE)
NEG = -1e30

_DN11 = (((1,), (1,)), ((), ()))
_DN00 = (((0,), (0,)), ((), ()))
_DN10 = (((1,), (0,)), ((), ()))
_DN01 = (((0,), (1,)), ((), ()))


def _moba_kernel(xq_ref, xk_ref, xv_ref, wq_ref, bq_ref, wk_ref, bk_ref,
                 wv_ref, bv_ref, wo_ref, bo_ref, o_ref, k_sc, v_sc):
    m = pl.program_id(0)

    @pl.when(m == 0)
    def _init():
        k_sc[:] = jnp.zeros((D_MODEL, S), jnp.float32)
        v_sc[:] = jnp.zeros((D_MODEL, S), jnp.float32)

    qt = jax.lax.dot_general(wq_ref[:], xq_ref[:], _DN11,
                             preferred_element_type=jnp.float32) + bq_ref[:]
    kt = jax.lax.dot_general(wk_ref[:], xk_ref[:], _DN11,
                             preferred_element_type=jnp.float32) + bk_ref[:]
    vt = jax.lax.dot_general(wv_ref[:], xv_ref[:], _DN11,
                             preferred_element_type=jnp.float32) + bv_ref[:]
    k_sc[:, pl.ds(m * BS, BS)] = kt
    v_sc[:, pl.ds(m * BS, BS)] = vt
    qs = qt * SCALE

    jidx = jax.lax.broadcasted_iota(jnp.int32, (NPRI, BS), 0)
    rows = jax.lax.broadcasted_iota(jnp.int32, (BS, BS), 0)
    cols = jax.lax.broadcasted_iota(jnp.int32, (BS, BS), 1)
    bd = (jax.lax.broadcasted_iota(jnp.int32, (NPRI, PRI), 1) // BS
          == jax.lax.broadcasted_iota(jnp.int32, (NPRI, PRI), 0)
          ).astype(jnp.float32)
    flag = jnp.where(m > 0, 1.0, 0.0).astype(jnp.float32)

    outs = []
    for h in range(H):
        lo = h * DH
        qh = qs[lo:lo + DH, :]

        k_pri = k_sc[lo:lo + DH, 0:PRI]
        v_pri = v_sc[lo:lo + DH, 0:PRI]
        s_all = jax.lax.dot_general(k_pri, qh, _DN00,
                                    preferred_element_type=jnp.float32)

        gate = jax.lax.dot_general(bd, s_all, _DN10,
                                   preferred_element_type=jnp.float32)
        gate = jnp.where(jidx < m, gate * INV_GATE, NEG)
        rank = jnp.zeros((NPRI, BS), jnp.int32)
        for jp in range(NPRI):
            gp = gate[jp:jp + 1, :]
            ahead = (gp > gate) | ((gp == gate) & (jp < jidx))
            rank = rank + ahead.astype(jnp.int32)
        sel = (rank < TOPK) & (jidx < m)

        k_i = k_sc[lo:lo + DH, pl.ds(m * BS, BS)]
        v_i = v_sc[lo:lo + DH, pl.ds(m * BS, BS)]
        s = jax.lax.dot_general(k_i, qh, _DN00,
                                preferred_element_type=jnp.float32)
        s = jnp.where(rows <= cols, s, NEG)
        m_self = jnp.max(s, axis=0, keepdims=True)
        p = jnp.exp(s - m_self)
        l_self = jnp.sum(p, axis=0, keepdims=True)
        o_self = jax.lax.dot_general(
            v_i, p, _DN10, preferred_element_type=jnp.float32) / l_self

        subs = [jnp.where(sel[j:j + 1, :], s_all[j * BS:(j + 1) * BS, :], NEG)
                for j in range(NPRI)]
        m_moba = jnp.full((1, BS), NEG, jnp.float32)
        for sub in subs:
            m_moba = jnp.maximum(m_moba, jnp.max(sub, axis=0, keepdims=True))
        p_all = jnp.concatenate([jnp.exp(sub - m_moba) for sub in subs],
                                axis=0)
        l = jnp.sum(p_all, axis=0, keepdims=True)
        acc = jax.lax.dot_general(v_pri, p_all, _DN10,
                                  preferred_element_type=jnp.float32)
        outs.append(o_self + (acc / l) * flag)

    combined = jnp.concatenate(outs, axis=0)
    o_ref[:] = jax.lax.dot_general(
        combined, wo_ref[:], _DN01,
        preferred_element_type=jnp.float32) + bo_ref[:]


def kernel(query, key, value, Wq, bq, Wk, bk, Wv, bv, Wo, bo):
    xq = query.reshape(S, D_MODEL)
    xk = key.reshape(S, D_MODEL)
    xv = value.reshape(S, D_MODEL)

    row_spec = pl.BlockSpec((BS, D_MODEL), lambda mm: (mm, 0))
    w_spec = pl.BlockSpec((D_MODEL, D_MODEL), lambda mm: (0, 0))
    b_spec = pl.BlockSpec((D_MODEL, 1), lambda mm: (0, 0))
    bo_spec = pl.BlockSpec((1, D_MODEL), lambda mm: (0, 0))

    out = pl.pallas_call(
        _moba_kernel,
        grid=(NB,),
        in_specs=[row_spec, row_spec, row_spec,
                  w_spec, b_spec, w_spec, b_spec, w_spec, b_spec,
                  w_spec, bo_spec],
        out_specs=row_spec,
        out_shape=jax.ShapeDtypeStruct((S, D_MODEL), jnp.float32),
        scratch_shapes=[
            pltpu.VMEM((D_MODEL, S), jnp.float32),
            pltpu.VMEM((D_MODEL, S), jnp.float32),
        ],
    )(xq, xk, xv, Wq, bq.reshape(-1, 1), Wk, bk.reshape(-1, 1),
      Wv, bv.reshape(-1, 1), Wo, bo.reshape(1, -1))

    return out.reshape(B, S, D_MODEL)


# speedup vs baseline: 1.0965x; 1.0965x over previous
import math

import jax
import jax.numpy as jnp
from jax.experimental import pallas as pl
from jax.experimental.pallas import tpu as pltpu

B = 1
S = 2048
D_MODEL = 768
H = 12
DH = D_MODEL // H
BS = 256
NB = S // BS
NPRI = NB - 1
PRI = NPRI * BS
TOPK = 3
SCALE = 1.0 / math.sqrt(DH)
NEG = -1e30

_DN = (((1,), (1,)), ((), ()))
_DN00 = (((0,), (0,)), ((), ()))
_DN01 = (((0,), (1,)), ((), ()))


def _moba_kernel(xq_ref, xk_ref, xv_ref, wq_ref, bq_ref, wk_ref, bk_ref,
                 wv_ref, bv_ref, wo_ref, bo_ref, o_ref,
                 k_sc, v_sc, km_sc):
    m = pl.program_id(0)

    @pl.when(m == 0)
    def _init():
        v_sc[:] = jnp.zeros((S, D_MODEL), jnp.float32)

    q = jax.lax.dot_general(xq_ref[:], wq_ref[:], _DN,
                            preferred_element_type=jnp.float32) + bq_ref[:]
    k = jax.lax.dot_general(xk_ref[:], wk_ref[:], _DN,
                            preferred_element_type=jnp.float32) + bk_ref[:]
    v = jax.lax.dot_general(xv_ref[:], wv_ref[:], _DN,
                            preferred_element_type=jnp.float32) + bv_ref[:]
    k_sc[pl.ds(m * BS, BS), :] = k
    v_sc[pl.ds(m * BS, BS), :] = v
    km_sc[pl.ds(m, 1), :] = jnp.mean(k, axis=0, keepdims=True)
    qs = q * SCALE

    jidx = jax.lax.broadcasted_iota(jnp.int32, (NPRI, BS), 0)
    rows = jax.lax.broadcasted_iota(jnp.int32, (BS, BS), 0)
    cols = jax.lax.broadcasted_iota(jnp.int32, (BS, BS), 1)
    flag = jnp.where(m > 0, 1.0, 0.0).astype(jnp.float32)

    outs = []
    for h in range(H):
        lo = h * DH
        qh = q[:, lo:lo + DH]
        qsh = qs[:, lo:lo + DH]
        km = km_sc[0:NPRI, lo:lo + DH]

        gate = jax.lax.dot_general(km, qh, _DN,
                                   preferred_element_type=jnp.float32)
        gate = jnp.where(jidx < m, gate, NEG)
        rank = jnp.zeros((NPRI, BS), jnp.int32)
        for jp in range(NPRI):
            gp = gate[jp:jp + 1, :]
            ahead = (gp > gate) | ((gp == gate) & (jp < jidx))
            rank = rank + ahead.astype(jnp.int32)
        sel = (rank < TOPK) & (jidx < m)

        k_i = k_sc[pl.ds(m * BS, BS), lo:lo + DH]
        v_i = v_sc[pl.ds(m * BS, BS), lo:lo + DH]
        s = jax.lax.dot_general(k_i, qsh, _DN,
                                preferred_element_type=jnp.float32)
        s = jnp.where(rows <= cols, s, NEG)
        m_self = jnp.max(s, axis=0, keepdims=True)
        p = jnp.exp(s - m_self)
        l_self = jnp.sum(p, axis=0, keepdims=True)
        o_self = jax.lax.dot_general(
            v_i, p, _DN00, preferred_element_type=jnp.float32) / l_self

        k_pri = k_sc[0:PRI, lo:lo + DH]
        v_pri = v_sc[0:PRI, lo:lo + DH]
        s_all = jax.lax.dot_general(k_pri, qsh, _DN,
                                    preferred_element_type=jnp.float32)
        subs = [jnp.where(sel[j:j + 1, :], s_all[j * BS:(j + 1) * BS, :], NEG)
                for j in range(NPRI)]
        m_moba = jnp.full((1, BS), NEG, jnp.float32)
        for sub in subs:
            m_moba = jnp.maximum(m_moba, jnp.max(sub, axis=0, keepdims=True))
        p_all = jnp.concatenate([jnp.exp(sub - m_moba) for sub in subs],
                                axis=0)
        l = jnp.sum(p_all, axis=0, keepdims=True)
        acc = jax.lax.dot_general(v_pri, p_all, _DN00,
                                  preferred_element_type=jnp.float32)
        outs.append(o_self + (acc / l) * flag)

    combined = jnp.concatenate(outs, axis=0)
    o_ref[:] = jax.lax.dot_general(
        combined, wo_ref[:], _DN01,
        preferred_element_type=jnp.float32) + bo_ref[:]


def kernel(query, key, value, Wq, bq, Wk, bk, Wv, bv, Wo, bo):
    xq = query.reshape(S, D_MODEL)
    xk = key.reshape(S, D_MODEL)
    xv = value.reshape(S, D_MODEL)

    row_spec = pl.BlockSpec((BS, D_MODEL), lambda mm: (mm, 0))
    w_spec = pl.BlockSpec((D_MODEL, D_MODEL), lambda mm: (0, 0))
    b_spec = pl.BlockSpec((1, D_MODEL), lambda mm: (0, 0))

    out = pl.pallas_call(
        _moba_kernel,
        grid=(NB,),
        in_specs=[row_spec, row_spec, row_spec,
                  w_spec, b_spec, w_spec, b_spec, w_spec, b_spec,
                  w_spec, b_spec],
        out_specs=row_spec,
        out_shape=jax.ShapeDtypeStruct((S, D_MODEL), jnp.float32),
        scratch_shapes=[
            pltpu.VMEM((S, D_MODEL), jnp.float32),
            pltpu.VMEM((S, D_MODEL), jnp.float32),
            pltpu.VMEM((NB, D_MODEL), jnp.float32),
        ],
    )(xq, xk, xv, Wq, bq.reshape(1, -1), Wk, bk.reshape(1, -1),
      Wv, bv.reshape(1, -1), Wo, bo.reshape(1, -1))

    return out.reshape(B, S, D_MODEL)


# speedup vs baseline: 4.2630x; 3.8879x over previous
import math

import jax
import jax.numpy as jnp
from jax.experimental import pallas as pl
from jax.experimental.pallas import tpu as pltpu

B = 1
S = 2048
D_MODEL = 768
H = 12
DH = D_MODEL // H
BS = 256
NB = S // BS
NPRI = NB - 1
PRI = NPRI * BS
TOPK = 3
SCALE = 1.0 / math.sqrt(DH)
NEG = -1e30

_DN = (((1,), (1,)), ((), ()))
_DN00 = (((0,), (0,)), ((), ()))
_DN01 = (((0,), (1,)), ((), ()))


def _moba_kernel(xq_ref, xk_ref, xv_ref, wq_ref, bq_ref, wk_ref, bk_ref,
                 wv_ref, bv_ref, wo_ref, bo_ref, o_ref,
                 k_sc, v_sc, km_sc):
    m = pl.program_id(0)

    @pl.when(m == 0)
    def _init():
        v_sc[:] = jnp.zeros((S, D_MODEL), jnp.float32)

    qs = (jax.lax.dot_general(xq_ref[:], wq_ref[:], _DN,
                              preferred_element_type=jnp.float32)
          + bq_ref[:]) * SCALE
    k = jax.lax.dot_general(xk_ref[:], wk_ref[:], _DN,
                            preferred_element_type=jnp.float32) + bk_ref[:]
    v = jax.lax.dot_general(xv_ref[:], wv_ref[:], _DN,
                            preferred_element_type=jnp.float32) + bv_ref[:]
    k_sc[pl.ds(m * BS, BS), :] = k
    v_sc[pl.ds(m * BS, BS), :] = v
    km_sc[pl.ds(m, 1), :] = jnp.mean(k, axis=0, keepdims=True)

    jidx = jax.lax.broadcasted_iota(jnp.int32, (NB, BS), 0)
    rows = jax.lax.broadcasted_iota(jnp.int32, (BS, BS), 0)
    cols = jax.lax.broadcasted_iota(jnp.int32, (BS, BS), 1)
    flag = jnp.where(m > 0, 1.0, 0.0).astype(jnp.float32)

    outs = []
    for h in range(H):
        lo = h * DH
        qsh = qs[:, lo:lo + DH]
        km = km_sc[:, lo:lo + DH]

        gate = jax.lax.dot_general(km, qsh, _DN,
                                   preferred_element_type=jnp.float32)
        gate = jnp.where(jidx < m, gate, NEG)
        rank = jnp.zeros((NB, BS), jnp.int32)
        for jp in range(NB):
            gp = gate[jp:jp + 1, :]
            ahead = (gp > gate) | ((gp == gate) & (jp < jidx))
            rank = rank + ahead.astype(jnp.int32)
        sel = ((rank < TOPK) & (jidx < m)).astype(jnp.float32)

        k_i = k_sc[pl.ds(m * BS, BS), lo:lo + DH]
        v_i = v_sc[pl.ds(m * BS, BS), lo:lo + DH]
        s = jax.lax.dot_general(k_i, qsh, _DN,
                                preferred_element_type=jnp.float32)
        s = jnp.where(rows <= cols, s, NEG)
        m_self = jnp.max(s, axis=0, keepdims=True)
        p = jnp.exp(s - m_self)
        l_self = jnp.sum(p, axis=0, keepdims=True)
        o_self = jax.lax.dot_general(
            v_i, p, _DN00, preferred_element_type=jnp.float32) / l_self

        k_pri = k_sc[0:PRI, lo:lo + DH]
        v_pri = v_sc[0:PRI, lo:lo + DH]
        s_all = jax.lax.dot_general(k_pri, qsh, _DN,
                                    preferred_element_type=jnp.float32)
        subs = [jnp.where(sel[j:j + 1, :] > 0.0,
                          s_all[j * BS:(j + 1) * BS, :], NEG)
                for j in range(NPRI)]
        m_moba = jnp.full((1, BS), NEG, jnp.float32)
        for sub in subs:
            m_moba = jnp.maximum(m_moba, jnp.max(sub, axis=0, keepdims=True))
        p_all = jnp.concatenate([jnp.exp(sub - m_moba) for sub in subs],
                                axis=0)
        l = jnp.sum(p_all, axis=0, keepdims=True)
        acc = jax.lax.dot_general(v_pri, p_all, _DN00,
                                  preferred_element_type=jnp.float32)
        outs.append(o_self + (acc / l) * flag)

    combined = jnp.concatenate(outs, axis=0)
    o_ref[:] = jax.lax.dot_general(
        combined, wo_ref[:], _DN01,
        preferred_element_type=jnp.float32) + bo_ref[:]


def kernel(query, key, value, Wq, bq, Wk, bk, Wv, bv, Wo, bo):
    xq = query.reshape(S, D_MODEL)
    xk = key.reshape(S, D_MODEL)
    xv = value.reshape(S, D_MODEL)

    row_spec = pl.BlockSpec((BS, D_MODEL), lambda mm: (mm, 0))
    w_spec = pl.BlockSpec((D_MODEL, D_MODEL), lambda mm: (0, 0))
    b_spec = pl.BlockSpec((1, D_MODEL), lambda mm: (0, 0))

    out = pl.pallas_call(
        _moba_kernel,
        grid=(NB,),
        in_specs=[row_spec, row_spec, row_spec,
                  w_spec, b_spec, w_spec, b_spec, w_spec, b_spec,
                  w_spec, b_spec],
        out_specs=row_spec,
        out_shape=jax.ShapeDtypeStruct((S, D_MODEL), jnp.float32),
        scratch_shapes=[
            pltpu.VMEM((S, D_MODEL), jnp.float32),
            pltpu.VMEM((S, D_MODEL), jnp.float32),
            pltpu.VMEM((NB, D_MODEL), jnp.float32),
        ],
    )(xq, xk, xv, Wq, bq.reshape(1, -1), Wk, bk.reshape(1, -1),
      Wv, bv.reshape(1, -1), Wo, bo.reshape(1, -1))

    return out.reshape(B, S, D_MODEL)


# speedup vs baseline: 4.3028x; 1.0094x over previous
import math

import jax
import jax.numpy as jnp
from jax.experimental import pallas as pl
from jax.experimental.pallas import tpu as pltpu

B = 1
S = 2048
D_MODEL = 768
H = 12
DH = D_MODEL // H
BS = 256
NB = S // BS
NPRI = NB - 1
PRI = NPRI * BS
TOPK = 3
SCALE = 1.0 / math.sqrt(DH)
NEG = -1e30

_DN = (((1,), (1,)), ((), ()))
_DN00 = (((0,), (0,)), ((), ()))
_DN01 = (((0,), (1,)), ((), ()))


def _moba_kernel(xq_ref, xk_ref, xv_ref, wq_ref, bq_ref, wk_ref, bk_ref,
                 wv_ref, bv_ref, wo_ref, bo_ref, o_ref,
                 k_sc, v_sc, km_sc):
    m = pl.program_id(0)

    @pl.when(m == 0)
    def _init():
        v_sc[:] = jnp.zeros((H, S, DH), jnp.float32)

    qs = (jax.lax.dot_general(xq_ref[:], wq_ref[:], _DN,
                              preferred_element_type=jnp.float32)
          + bq_ref[:]) * SCALE
    k = jax.lax.dot_general(xk_ref[:], wk_ref[:], _DN,
                            preferred_element_type=jnp.float32) + bk_ref[:]
    v = jax.lax.dot_general(xv_ref[:], wv_ref[:], _DN,
                            preferred_element_type=jnp.float32) + bv_ref[:]
    for h in range(H):
        k_sc[h, pl.ds(m * BS, BS), :] = k[:, h * DH:(h + 1) * DH]
        v_sc[h, pl.ds(m * BS, BS), :] = v[:, h * DH:(h + 1) * DH]
    km_sc[pl.ds(m, 1), :] = jnp.mean(k, axis=0, keepdims=True)

    jidx = jax.lax.broadcasted_iota(jnp.int32, (NB, BS), 0)
    rows = jax.lax.broadcasted_iota(jnp.int32, (BS, BS), 0)
    cols = jax.lax.broadcasted_iota(jnp.int32, (BS, BS), 1)
    flag = jnp.where(m > 0, 1.0, 0.0).astype(jnp.float32)

    outs = []
    for h in range(H):
        lo = h * DH
        qsh = qs[:, lo:lo + DH]
        km = km_sc[:, lo:lo + DH]

        gate = jax.lax.dot_general(km, qsh, _DN,
                                   preferred_element_type=jnp.float32)
        gate = jnp.where(jidx < m, gate, NEG)
        rank = jnp.zeros((NB, BS), jnp.int32)
        for jp in range(NB):
            gp = gate[jp:jp + 1, :]
            ahead = (gp > gate) | ((gp == gate) & (jp < jidx))
            rank = rank + ahead.astype(jnp.int32)
        sel = ((rank < TOPK) & (jidx < m)).astype(jnp.float32)

        k_i = k_sc[h, pl.ds(m * BS, BS), :]
        v_i = v_sc[h, pl.ds(m * BS, BS), :]
        s = jax.lax.dot_general(k_i, qsh, _DN,
                                preferred_element_type=jnp.float32)
        s = jnp.where(rows <= cols, s, NEG)
        m_self = jnp.max(s, axis=0, keepdims=True)
        p = jnp.exp(s - m_self)
        l_self = jnp.sum(p, axis=0, keepdims=True)
        o_self = jax.lax.dot_general(
            v_i, p, _DN00, preferred_element_type=jnp.float32) / l_self

        k_pri = k_sc[h, 0:PRI, :]
        v_pri = v_sc[h, 0:PRI, :]
        s_all = jax.lax.dot_general(k_pri, qsh, _DN,
                                    preferred_element_type=jnp.float32)
        subs = [jnp.where(sel[j:j + 1, :] > 0.0,
                          s_all[j * BS:(j + 1) * BS, :], NEG)
                for j in range(NPRI)]
        m_moba = jnp.full((1, BS), NEG, jnp.float32)
        for sub in subs:
            m_moba = jnp.maximum(m_moba, jnp.max(sub, axis=0, keepdims=True))
        p_all = jnp.concatenate([jnp.exp(sub - m_moba) for sub in subs],
                                axis=0)
        l = jnp.sum(p_all, axis=0, keepdims=True)
        acc = jax.lax.dot_general(v_pri, p_all, _DN00,
                                  preferred_element_type=jnp.float32)
        outs.append(o_self + (acc / l) * flag)

    combined = jnp.concatenate(outs, axis=0)
    o_ref[:] = jax.lax.dot_general(
        combined, wo_ref[:], _DN01,
        preferred_element_type=jnp.float32) + bo_ref[:]


def kernel(query, key, value, Wq, bq, Wk, bk, Wv, bv, Wo, bo):
    xq = query.reshape(S, D_MODEL)
    xk = key.reshape(S, D_MODEL)
    xv = value.reshape(S, D_MODEL)

    row_spec = pl.BlockSpec((BS, D_MODEL), lambda mm: (mm, 0))
    w_spec = pl.BlockSpec((D_MODEL, D_MODEL), lambda mm: (0, 0))
    b_spec = pl.BlockSpec((1, D_MODEL), lambda mm: (0, 0))

    out = pl.pallas_call(
        _moba_kernel,
        grid=(NB,),
        in_specs=[row_spec, row_spec, row_spec,
                  w_spec, b_spec, w_spec, b_spec, w_spec, b_spec,
                  w_spec, b_spec],
        out_specs=row_spec,
        out_shape=jax.ShapeDtypeStruct((S, D_MODEL), jnp.float32),
        scratch_shapes=[
            pltpu.VMEM((H, S, DH), jnp.float32),
            pltpu.VMEM((H, S, DH), jnp.float32),
            pltpu.VMEM((NB, D_MODEL), jnp.float32),
        ],
    )(xq, xk, xv, Wq, bq.reshape(1, -1), Wk, bk.reshape(1, -1),
      Wv, bv.reshape(1, -1), Wo, bo.reshape(1, -1))

    return out.reshape(B, S, D_MODEL)
